# baseline (speedup 1.0000x reference)
import jax
import jax.numpy as jnp
from jax import lax
from jax.experimental import pallas as pl
from jax.experimental.pallas import tpu as pltpu

N, D = 100000, 128
B = 4096
FULL_STEPS = N // B
TB = N - FULL_STEPS * B
K = 6


def _scalar_math(max_a, his_m, agg_n, inv_w):
    beta = jnp.clip(1.0 - inv_w * agg_n, 0.0, 1.0)
    max_m = jnp.maximum(max_a, his_m)
    neg_inf = jnp.float32(-jnp.inf)
    dp = his_m - max_m
    dq = max_a - max_m
    dp = jnp.where(jnp.isnan(dp), neg_inf, dp)
    dq = jnp.where(jnp.isnan(dq), neg_inf, dq)
    p = jnp.exp(dp) * beta
    q = jnp.exp(dq)
    t = jnp.maximum(p + q, 1.0)
    inv_t = 1.0 / t
    return (p * inv_t)[:, None], (q * inv_t)[:, None]


def _body(x_hbm, ma_hbm, an_hbm, hm_hbm, iw_hbm, hx_hbm, out_hbm,
          xb, hb, ob, xt, ht, ot, mav, anv, hmv, iwv,
          load_sem, store_sem, scal_sem, tail_sem, tstore_sem):

    def start_load(step, slot):
        rows = pl.ds(step * B, B)
        pltpu.make_async_copy(x_hbm.at[rows], xb.at[slot], load_sem.at[slot, 0]).start()
        pltpu.make_async_copy(hx_hbm.at[rows], hb.at[slot], load_sem.at[slot, 1]).start()

    def wait_load(step, slot):
        rows = pl.ds(step * B, B)
        pltpu.make_async_copy(x_hbm.at[rows], xb.at[slot], load_sem.at[slot, 0]).wait()
        pltpu.make_async_copy(hx_hbm.at[rows], hb.at[slot], load_sem.at[slot, 1]).wait()

    pltpu.make_async_copy(ma_hbm, mav, scal_sem.at[0]).start()
    pltpu.make_async_copy(an_hbm, anv, scal_sem.at[1]).start()
    pltpu.make_async_copy(hm_hbm, hmv, scal_sem.at[2]).start()
    pltpu.make_async_copy(iw_hbm, iwv, scal_sem.at[3]).start()
    tail_rows = pl.ds(FULL_STEPS * B, TB)
    pltpu.make_async_copy(x_hbm.at[tail_rows], xt, tail_sem.at[0]).start()
    pltpu.make_async_copy(hx_hbm.at[tail_rows], ht, tail_sem.at[1]).start()
    for s in range(K - 1):
        start_load(s, s)

    pltpu.make_async_copy(ma_hbm, mav, scal_sem.at[0]).wait()
    pltpu.make_async_copy(an_hbm, anv, scal_sem.at[1]).wait()
    pltpu.make_async_copy(hm_hbm, hmv, scal_sem.at[2]).wait()
    pltpu.make_async_copy(iw_hbm, iwv, scal_sem.at[3]).wait()

    pltpu.make_async_copy(x_hbm.at[tail_rows], xt, tail_sem.at[0]).wait()
    pltpu.make_async_copy(hx_hbm.at[tail_rows], ht, tail_sem.at[1]).wait()
    tsl = pl.ds(FULL_STEPS * B, TB)
    p2, q2 = _scalar_math(mav[tsl], hmv[tsl], anv[tsl], iwv[tsl])
    ot[...] = ht[...] * p2 + xt[...] * q2
    pltpu.make_async_copy(ot, out_hbm.at[tail_rows], tstore_sem).start()

    def step_fn(i, carry):
        slot = lax.rem(i, K)

        @pl.when(i >= K)
        def _():
            rows_old = pl.ds((i - K) * B, B)
            pltpu.make_async_copy(ob.at[slot], out_hbm.at[rows_old],
                                  store_sem.at[slot]).wait()

        @pl.when(i + K - 1 < FULL_STEPS)
        def _():
            start_load(i + K - 1, lax.rem(i + K - 1, K))

        wait_load(i, slot)

        off = pl.multiple_of(i * B, B)
        sl = pl.ds(off, B)
        p2, q2 = _scalar_math(mav[sl], hmv[sl], anv[sl], iwv[sl])
        ob[slot] = hb[slot] * p2 + xb[slot] * q2

        rows = pl.ds(i * B, B)
        pltpu.make_async_copy(ob.at[slot], out_hbm.at[rows],
                              store_sem.at[slot]).start()
        return carry

    lax.fori_loop(0, FULL_STEPS, step_fn, 0)

    for i in range(FULL_STEPS - K, FULL_STEPS):
        slot = i % K
        rows = pl.ds(i * B, B)
        pltpu.make_async_copy(ob.at[slot], out_hbm.at[rows],
                              store_sem.at[slot]).wait()
    pltpu.make_async_copy(ot, out_hbm.at[tail_rows], tstore_sem).wait()


def kernel(x, max_a, agg_n, his_x, his_m, inv_w):
    any_spec = pl.BlockSpec(memory_space=pl.ANY)
    return pl.pallas_call(
        _body,
        in_specs=[any_spec] * 6,
        out_specs=any_spec,
        out_shape=jax.ShapeDtypeStruct((N, D), jnp.float32),
        scratch_shapes=[
            pltpu.VMEM((K, B, D), jnp.float32),
            pltpu.VMEM((K, B, D), jnp.float32),
            pltpu.VMEM((K, B, D), jnp.float32),
            pltpu.VMEM((TB, D), jnp.float32),
            pltpu.VMEM((TB, D), jnp.float32),
            pltpu.VMEM((TB, D), jnp.float32),
            pltpu.VMEM((N,), jnp.float32),
            pltpu.VMEM((N,), jnp.float32),
            pltpu.VMEM((N,), jnp.float32),
            pltpu.VMEM((N,), jnp.float32),
            pltpu.SemaphoreType.DMA((K, 2)),
            pltpu.SemaphoreType.DMA((K,)),
            pltpu.SemaphoreType.DMA((4,)),
            pltpu.SemaphoreType.DMA((2,)),
            pltpu.SemaphoreType.DMA,
        ],
    )(x, max_a, agg_n, his_m, inv_w, his_x)


# speedup vs baseline: 1.0027x; 1.0027x over previous
import jax
import jax.numpy as jnp
from jax import lax
from jax.experimental import pallas as pl
from jax.experimental.pallas import tpu as pltpu

N, D = 100000, 128
B = 8192
FULL_STEPS = N // B
TB = N - FULL_STEPS * B
K = 3


def _scalar_math(max_a, his_m, agg_n, inv_w):
    beta = jnp.clip(1.0 - inv_w * agg_n, 0.0, 1.0)
    max_m = jnp.maximum(max_a, his_m)
    neg_inf = jnp.float32(-jnp.inf)
    dp = his_m - max_m
    dq = max_a - max_m
    dp = jnp.where(jnp.isnan(dp), neg_inf, dp)
    dq = jnp.where(jnp.isnan(dq), neg_inf, dq)
    p = jnp.exp(dp) * beta
    q = jnp.exp(dq)
    t = jnp.maximum(p + q, 1.0)
    inv_t = 1.0 / t
    return (p * inv_t)[:, None], (q * inv_t)[:, None]


def _body(x_hbm, ma_hbm, an_hbm, hm_hbm, iw_hbm, hx_hbm, out_hbm,
          xb, hb, ob, xt, ht, ot, mav, anv, hmv, iwv,
          load_sem, store_sem, scal_sem, tail_sem, tstore_sem):

    def start_load(step, slot):
        rows = pl.ds(step * B, B)
        pltpu.make_async_copy(x_hbm.at[rows], xb.at[slot], load_sem.at[slot, 0]).start()
        pltpu.make_async_copy(hx_hbm.at[rows], hb.at[slot], load_sem.at[slot, 1]).start()

    def wait_load(step, slot):
        rows = pl.ds(step * B, B)
        pltpu.make_async_copy(x_hbm.at[rows], xb.at[slot], load_sem.at[slot, 0]).wait()
        pltpu.make_async_copy(hx_hbm.at[rows], hb.at[slot], load_sem.at[slot, 1]).wait()

    pltpu.make_async_copy(ma_hbm, mav, scal_sem.at[0]).start()
    pltpu.make_async_copy(an_hbm, anv, scal_sem.at[1]).start()
    pltpu.make_async_copy(hm_hbm, hmv, scal_sem.at[2]).start()
    pltpu.make_async_copy(iw_hbm, iwv, scal_sem.at[3]).start()
    tail_rows = pl.ds(FULL_STEPS * B, TB)
    pltpu.make_async_copy(x_hbm.at[tail_rows], xt, tail_sem.at[0]).start()
    pltpu.make_async_copy(hx_hbm.at[tail_rows], ht, tail_sem.at[1]).start()
    for s in range(K - 1):
        start_load(s, s)

    pltpu.make_async_copy(ma_hbm, mav, scal_sem.at[0]).wait()
    pltpu.make_async_copy(an_hbm, anv, scal_sem.at[1]).wait()
    pltpu.make_async_copy(hm_hbm, hmv, scal_sem.at[2]).wait()
    pltpu.make_async_copy(iw_hbm, iwv, scal_sem.at[3]).wait()

    pltpu.make_async_copy(x_hbm.at[tail_rows], xt, tail_sem.at[0]).wait()
    pltpu.make_async_copy(hx_hbm.at[tail_rows], ht, tail_sem.at[1]).wait()
    tsl = pl.ds(FULL_STEPS * B, TB)
    p2, q2 = _scalar_math(mav[tsl], hmv[tsl], anv[tsl], iwv[tsl])
    ot[...] = ht[...] * p2 + xt[...] * q2
    pltpu.make_async_copy(ot, out_hbm.at[tail_rows], tstore_sem).start()

    def step_fn(i, carry):
        slot = lax.rem(i, K)

        @pl.when(i >= K)
        def _():
            rows_old = pl.ds((i - K) * B, B)
            pltpu.make_async_copy(ob.at[slot], out_hbm.at[rows_old],
                                  store_sem.at[slot]).wait()

        @pl.when(i + K - 1 < FULL_STEPS)
        def _():
            start_load(i + K - 1, lax.rem(i + K - 1, K))

        wait_load(i, slot)

        off = pl.multiple_of(i * B, B)
        sl = pl.ds(off, B)
        p2, q2 = _scalar_math(mav[sl], hmv[sl], anv[sl], iwv[sl])
        ob[slot] = hb[slot] * p2 + xb[slot] * q2

        rows = pl.ds(i * B, B)
        pltpu.make_async_copy(ob.at[slot], out_hbm.at[rows],
                              store_sem.at[slot]).start()
        return carry

    lax.fori_loop(0, FULL_STEPS, step_fn, 0)

    for i in range(FULL_STEPS - K, FULL_STEPS):
        slot = i % K
        rows = pl.ds(i * B, B)
        pltpu.make_async_copy(ob.at[slot], out_hbm.at[rows],
                              store_sem.at[slot]).wait()
    pltpu.make_async_copy(ot, out_hbm.at[tail_rows], tstore_sem).wait()


def kernel(x, max_a, agg_n, his_x, his_m, inv_w):
    any_spec = pl.BlockSpec(memory_space=pl.ANY)
    return pl.pallas_call(
        _body,
        in_specs=[any_spec] * 6,
        out_specs=any_spec,
        out_shape=jax.ShapeDtypeStruct((N, D), jnp.float32),
        scratch_shapes=[
            pltpu.VMEM((K, B, D), jnp.float32),
            pltpu.VMEM((K, B, D), jnp.float32),
            pltpu.VMEM((K, B, D), jnp.float32),
            pltpu.VMEM((TB, D), jnp.float32),
            pltpu.VMEM((TB, D), jnp.float32),
            pltpu.VMEM((TB, D), jnp.float32),
            pltpu.VMEM((N,), jnp.float32),
            pltpu.VMEM((N,), jnp.float32),
            pltpu.VMEM((N,), jnp.float32),
            pltpu.VMEM((N,), jnp.float32),
            pltpu.SemaphoreType.DMA((K, 2)),
            pltpu.SemaphoreType.DMA((K,)),
            pltpu.SemaphoreType.DMA((4,)),
            pltpu.SemaphoreType.DMA((2,)),
            pltpu.SemaphoreType.DMA,
        ],
    )(x, max_a, agg_n, his_m, inv_w, his_x)


# speedup vs baseline: 1.0065x; 1.0038x over previous
import jax
import jax.numpy as jnp
from jax import lax
from jax.experimental import pallas as pl
from jax.experimental.pallas import tpu as pltpu

N, D = 100000, 128
B = 8192
FULL_STEPS = N // B
TB = N - FULL_STEPS * B
K = 4


def _scalar_math(max_a, his_m, agg_n, inv_w):
    beta = jnp.clip(1.0 - inv_w * agg_n, 0.0, 1.0)
    max_m = jnp.maximum(max_a, his_m)
    neg_inf = jnp.float32(-jnp.inf)
    dp = his_m - max_m
    dq = max_a - max_m
    dp = jnp.where(jnp.isnan(dp), neg_inf, dp)
    dq = jnp.where(jnp.isnan(dq), neg_inf, dq)
    p = jnp.exp(dp) * beta
    q = jnp.exp(dq)
    t = jnp.maximum(p + q, 1.0)
    inv_t = 1.0 / t
    return (p * inv_t)[:, None], (q * inv_t)[:, None]


def _body(x_hbm, ma_hbm, an_hbm, hm_hbm, iw_hbm, hx_hbm, out_hbm,
          xb, hb, ob, xt, ht, ot, mav, anv, hmv, iwv,
          load_sem, store_sem, scal_sem, tail_sem, tstore_sem):

    def start_load(step, slot):
        rows = pl.ds(step * B, B)
        pltpu.make_async_copy(x_hbm.at[rows], xb.at[slot], load_sem.at[slot, 0]).start()
        pltpu.make_async_copy(hx_hbm.at[rows], hb.at[slot], load_sem.at[slot, 1]).start()

    def wait_load(step, slot):
        rows = pl.ds(step * B, B)
        pltpu.make_async_copy(x_hbm.at[rows], xb.at[slot], load_sem.at[slot, 0]).wait()
        pltpu.make_async_copy(hx_hbm.at[rows], hb.at[slot], load_sem.at[slot, 1]).wait()

    pltpu.make_async_copy(ma_hbm, mav, scal_sem.at[0]).start()
    pltpu.make_async_copy(an_hbm, anv, scal_sem.at[1]).start()
    pltpu.make_async_copy(hm_hbm, hmv, scal_sem.at[2]).start()
    pltpu.make_async_copy(iw_hbm, iwv, scal_sem.at[3]).start()
    tail_rows = pl.ds(FULL_STEPS * B, TB)
    pltpu.make_async_copy(x_hbm.at[tail_rows], xt, tail_sem.at[0]).start()
    pltpu.make_async_copy(hx_hbm.at[tail_rows], ht, tail_sem.at[1]).start()
    for s in range(K - 1):
        start_load(s, s)

    pltpu.make_async_copy(ma_hbm, mav, scal_sem.at[0]).wait()
    pltpu.make_async_copy(an_hbm, anv, scal_sem.at[1]).wait()
    pltpu.make_async_copy(hm_hbm, hmv, scal_sem.at[2]).wait()
    pltpu.make_async_copy(iw_hbm, iwv, scal_sem.at[3]).wait()

    pltpu.make_async_copy(x_hbm.at[tail_rows], xt, tail_sem.at[0]).wait()
    pltpu.make_async_copy(hx_hbm.at[tail_rows], ht, tail_sem.at[1]).wait()
    tsl = pl.ds(FULL_STEPS * B, TB)
    p2, q2 = _scalar_math(mav[tsl], hmv[tsl], anv[tsl], iwv[tsl])
    ot[...] = ht[...] * p2 + xt[...] * q2
    pltpu.make_async_copy(ot, out_hbm.at[tail_rows], tstore_sem).start()

    def step_fn(i, carry):
        slot = lax.rem(i, K)

        @pl.when(i >= K)
        def _():
            rows_old = pl.ds((i - K) * B, B)
            pltpu.make_async_copy(ob.at[slot], out_hbm.at[rows_old],
                                  store_sem.at[slot]).wait()

        @pl.when(i + K - 1 < FULL_STEPS)
        def _():
            start_load(i + K - 1, lax.rem(i + K - 1, K))

        wait_load(i, slot)

        off = pl.multiple_of(i * B, B)
        sl = pl.ds(off, B)
        p2, q2 = _scalar_math(mav[sl], hmv[sl], anv[sl], iwv[sl])
        ob[slot] = hb[slot] * p2 + xb[slot] * q2

        rows = pl.ds(i * B, B)
        pltpu.make_async_copy(ob.at[slot], out_hbm.at[rows],
                              store_sem.at[slot]).start()
        return carry

    lax.fori_loop(0, FULL_STEPS, step_fn, 0)

    for i in range(FULL_STEPS - K, FULL_STEPS):
        slot = i % K
        rows = pl.ds(i * B, B)
        pltpu.make_async_copy(ob.at[slot], out_hbm.at[rows],
                              store_sem.at[slot]).wait()
    pltpu.make_async_copy(ot, out_hbm.at[tail_rows], tstore_sem).wait()


def kernel(x, max_a, agg_n, his_x, his_m, inv_w):
    any_spec = pl.BlockSpec(memory_space=pl.ANY)
    return pl.pallas_call(
        _body,
        in_specs=[any_spec] * 6,
        out_specs=any_spec,
        out_shape=jax.ShapeDtypeStruct((N, D), jnp.float32),
        scratch_shapes=[
            pltpu.VMEM((K, B, D), jnp.float32),
            pltpu.VMEM((K, B, D), jnp.float32),
            pltpu.VMEM((K, B, D), jnp.float32),
            pltpu.VMEM((TB, D), jnp.float32),
            pltpu.VMEM((TB, D), jnp.float32),
            pltpu.VMEM((TB, D), jnp.float32),
            pltpu.VMEM((N,), jnp.float32),
            pltpu.VMEM((N,), jnp.float32),
            pltpu.VMEM((N,), jnp.float32),
            pltpu.VMEM((N,), jnp.float32),
            pltpu.SemaphoreType.DMA((K, 2)),
            pltpu.SemaphoreType.DMA((K,)),
            pltpu.SemaphoreType.DMA((4,)),
            pltpu.SemaphoreType.DMA((2,)),
            pltpu.SemaphoreType.DMA,
        ],
    )(x, max_a, agg_n, his_m, inv_w, his_x)
